# baseline (speedup 1.0000x reference)
import functools

import jax
import jax.numpy as jnp
from jax import lax
from jax.experimental import pallas as pl
from jax.experimental.pallas import tpu as pltpu
from jax.experimental.pallas import tpu_sc as plsc

BATCH = 128
ROW = 8192
LANES = 16
NUM_CORES = 2
NUM_SUBCORES = 16
NUM_WORKERS = NUM_CORES * NUM_SUBCORES
ROWS_PER_W = BATCH // NUM_WORKERS
NCHAINS = 4
QCOLS = ROW // NCHAINS
QCHUNKS = QCOLS // LANES
UNROLL = 4


def _sc_body(score_hbm, ans_hbm, out_hbm, s_v, g_v, st_v, gsem, *rsems):
    wid = lax.axis_index("s") * NUM_CORES + lax.axis_index("c")
    base = wid * ROWS_PER_W

    row_copies = [
        pltpu.async_copy(
            score_hbm.at[pl.ds(base + r, 1)], s_v.at[pl.ds(r, 1)], rsems[r]
        )
        for r in range(ROWS_PER_W)
    ]

    iota = lax.iota(jnp.int32, LANES)
    ones = jnp.ones((LANES,), jnp.int32)
    partial = jnp.zeros((LANES,), jnp.float32)
    gathers = []
    lane_sels = []
    for r in range(ROWS_PER_W):
        row_copies[r].wait()

        def chunk(c, carry, r=r):
            vmax, vchunk, vcnt = carry
            for u in range(UNROLL):
                nmax, nchunk = [], []
                for q in range(NCHAINS):
                    s = s_v[r, pl.ds(c * (UNROLL * LANES) + q * QCOLS + u * LANES, LANES)]
                    pred = s > vmax[q]
                    nmax.append(jnp.where(pred, s, vmax[q]))
                    nchunk.append(jnp.where(pred, vcnt, vchunk[q]))
                vmax, vchunk = nmax, nchunk
                vcnt = vcnt + ones
            return vmax, vchunk, vcnt

        init = (
            [jnp.full((LANES,), -jnp.inf, jnp.float32)] * NCHAINS,
            [jnp.zeros((LANES,), jnp.int32)] * NCHAINS,
            jnp.zeros((LANES,), jnp.int32),
        )
        vmax, vchunk, _ = lax.fori_loop(0, QCHUNKS // UNROLL, chunk, init)
        bm = vmax[0]
        bi = vchunk[0] * LANES + iota
        for q in range(1, NCHAINS):
            vidx = vchunk[q] * LANES + (q * QCOLS) + iota
            p = vmax[q] > bm
            bm = jnp.where(p, vmax[q], bm)
            bi = jnp.where(p, vidx, bi)
        m = jnp.max(bm)
        cand = jnp.where(bm == m, bi, jnp.int32(1 << 30))
        imin = jnp.min(cand)
        col0 = pl.multiple_of(jnp.bitwise_and(imin, jnp.int32(-128)), 128)
        off = imin - col0
        lane_sels.append(off)
        gathers.append(
            pltpu.async_copy(
                ans_hbm.at[pl.ds(base + r, 1), pl.ds(col0, 128)],
                g_v.at[pl.ds(r, 1)],
                gsem,
            )
        )

    for r in range(ROWS_PER_W):
        gathers[r].wait()
        off = lane_sels[r]
        sub = pl.multiple_of(jnp.bitwise_and(off, jnp.int32(-LANES)), LANES)
        gv = g_v[r, pl.ds(sub, LANES)]
        lane = jnp.bitwise_and(off, jnp.int32(LANES - 1))
        partial = partial + jnp.where(iota == lane, gv, jnp.float32(0.0))

    st_v[...] = partial
    pltpu.sync_copy(st_v, out_hbm.at[wid])


@jax.jit
def _sc_partials(score, ans_idx):
    mesh = plsc.VectorSubcoreMesh(core_axis_name="c", subcore_axis_name="s")
    return pl.kernel(
        _sc_body,
        out_type=jax.ShapeDtypeStruct((NUM_WORKERS, LANES), jnp.float32),
        mesh=mesh,
        scratch_types=[
            pltpu.VMEM((ROWS_PER_W, ROW), jnp.float32),
            pltpu.VMEM((ROWS_PER_W, 128), jnp.float32),
            pltpu.VMEM((LANES,), jnp.float32),
            pltpu.SemaphoreType.DMA,
        ]
        + [pltpu.SemaphoreType.DMA] * ROWS_PER_W,
        compiler_params=pltpu.CompilerParams(
            needs_layout_passes=False, use_tc_tiling_on_sc=False
        ),
    )(score, ans_idx)


def _reduce_body(p_ref, o_ref):
    o_ref[0, 0] = jnp.sum(p_ref[...]) * (100.0 / BATCH)


@jax.jit
def _tc_reduce(partials):
    return pl.pallas_call(
        _reduce_body,
        out_shape=jax.ShapeDtypeStruct((1, 1), jnp.float32),
        in_specs=[pl.BlockSpec(memory_space=pltpu.VMEM)],
        out_specs=pl.BlockSpec(memory_space=pltpu.SMEM),
    )(partials)


def kernel(score, ans_idx):
    partials = _sc_partials(score, ans_idx)
    acc = _tc_reduce(partials)
    return acc[0, 0]


# speedup vs baseline: 1.2247x; 1.2247x over previous
import functools

import jax
import jax.numpy as jnp
from jax import lax
from jax.experimental import pallas as pl
from jax.experimental.pallas import tpu as pltpu
from jax.experimental.pallas import tpu_sc as plsc

BATCH = 128
ROW = 8192
LANES = 16
NUM_CORES = 2
NUM_SUBCORES = 16
NUM_WORKERS = NUM_CORES * NUM_SUBCORES
ROWS_PER_W = BATCH // NUM_WORKERS
NCHAINS = 4
QCOLS = ROW // NCHAINS
QCHUNKS = QCOLS // LANES
UNROLL = 4


def _sc_body(score_hbm, ans_hbm, out_hbm, s_v, a_v, st_v, asem):
    wid = lax.axis_index("s") * NUM_CORES + lax.axis_index("c")
    base = wid * ROWS_PER_W
    pltpu.sync_copy(score_hbm.at[pl.ds(base, ROWS_PER_W)], s_v)
    ans_copy = pltpu.async_copy(ans_hbm.at[pl.ds(base, ROWS_PER_W)], a_v, asem)

    iota = lax.iota(jnp.int32, LANES)
    ones = jnp.ones((LANES,), jnp.int32)
    imins = []
    for r in range(ROWS_PER_W):
        def chunk(c, carry, r=r):
            vmax, vchunk, vcnt = carry
            for u in range(UNROLL):
                nmax, nchunk = [], []
                for q in range(NCHAINS):
                    s = s_v[r, pl.ds(c * (UNROLL * LANES) + q * QCOLS + u * LANES, LANES)]
                    pred = s > vmax[q]
                    nmax.append(jnp.where(pred, s, vmax[q]))
                    nchunk.append(jnp.where(pred, vcnt, vchunk[q]))
                vmax, vchunk = nmax, nchunk
                vcnt = vcnt + ones
            return vmax, vchunk, vcnt

        init = (
            [jnp.full((LANES,), -jnp.inf, jnp.float32)] * NCHAINS,
            [jnp.zeros((LANES,), jnp.int32)] * NCHAINS,
            jnp.zeros((LANES,), jnp.int32),
        )
        vmax, vchunk, _ = lax.fori_loop(0, QCHUNKS // UNROLL, chunk, init)
        bm = vmax[0]
        bi = vchunk[0] * LANES + iota
        for q in range(1, NCHAINS):
            vidx = vchunk[q] * LANES + (q * QCOLS) + iota
            p = vmax[q] > bm
            bm = jnp.where(p, vmax[q], bm)
            bi = jnp.where(p, vidx, bi)
        m = jnp.max(bm)
        cand = jnp.where(bm == m, bi, jnp.int32(1 << 30))
        imins.append(jnp.min(cand))

    ans_copy.wait()
    partial = jnp.zeros((LANES,), jnp.float32)
    for r in range(ROWS_PER_W):
        imin = imins[r]
        col0 = pl.multiple_of(jnp.bitwise_and(imin, jnp.int32(-LANES)), LANES)
        av = a_v[r, pl.ds(col0, LANES)]
        lane = jnp.bitwise_and(imin, jnp.int32(LANES - 1))
        partial = partial + jnp.where(iota == lane, av, jnp.float32(0.0))

    st_v[...] = partial
    pltpu.sync_copy(st_v, out_hbm.at[wid])


@jax.jit
def _sc_partials(score, ans_idx):
    mesh = plsc.VectorSubcoreMesh(core_axis_name="c", subcore_axis_name="s")
    return pl.kernel(
        _sc_body,
        out_type=jax.ShapeDtypeStruct((NUM_WORKERS, LANES), jnp.float32),
        mesh=mesh,
        scratch_types=[
            pltpu.VMEM((ROWS_PER_W, ROW), jnp.float32),
            pltpu.VMEM((ROWS_PER_W, ROW), jnp.float32),
            pltpu.VMEM((LANES,), jnp.float32),
            pltpu.SemaphoreType.DMA,
        ],
        compiler_params=pltpu.CompilerParams(needs_layout_passes=False),
    )(score, ans_idx)


def _reduce_body(p_ref, o_ref):
    o_ref[0, 0] = jnp.sum(p_ref[...]) * (100.0 / BATCH)


@jax.jit
def _tc_reduce(partials):
    return pl.pallas_call(
        _reduce_body,
        out_shape=jax.ShapeDtypeStruct((1, 1), jnp.float32),
        in_specs=[pl.BlockSpec(memory_space=pltpu.VMEM)],
        out_specs=pl.BlockSpec(memory_space=pltpu.SMEM),
    )(partials)


def kernel(score, ans_idx):
    partials = _sc_partials(score, ans_idx)
    acc = _tc_reduce(partials)
    return acc[0, 0]


# speedup vs baseline: 1.2377x; 1.0106x over previous
import functools

import jax
import jax.numpy as jnp
from jax import lax
from jax.experimental import pallas as pl
from jax.experimental.pallas import tpu as pltpu
from jax.experimental.pallas import tpu_sc as plsc

BATCH = 128
ROW = 8192
LANES = 16
NUM_CORES = 2
NUM_SUBCORES = 16
NUM_WORKERS = NUM_CORES * NUM_SUBCORES
ROWS_PER_W = BATCH // NUM_WORKERS
NCHAINS = 8
QCOLS = ROW // NCHAINS
QCHUNKS = QCOLS // LANES
UNROLL = 2


def _sc_body(score_hbm, ans_hbm, out_hbm, s_v, a_v, st_v, asem):
    wid = lax.axis_index("s") * NUM_CORES + lax.axis_index("c")
    base = wid * ROWS_PER_W
    pltpu.sync_copy(score_hbm.at[pl.ds(base, ROWS_PER_W)], s_v)
    ans_copy = pltpu.async_copy(ans_hbm.at[pl.ds(base, ROWS_PER_W)], a_v, asem)

    iota = lax.iota(jnp.int32, LANES)
    ones = jnp.ones((LANES,), jnp.int32)
    imins = []
    for r in range(ROWS_PER_W):
        def chunk(c, carry, r=r):
            vmax, vchunk, vcnt = carry
            for u in range(UNROLL):
                nmax, nchunk = [], []
                for q in range(NCHAINS):
                    s = s_v[r, pl.ds(c * (UNROLL * LANES) + q * QCOLS + u * LANES, LANES)]
                    pred = s > vmax[q]
                    nmax.append(jnp.where(pred, s, vmax[q]))
                    nchunk.append(jnp.where(pred, vcnt, vchunk[q]))
                vmax, vchunk = nmax, nchunk
                vcnt = vcnt + ones
            return vmax, vchunk, vcnt

        init = (
            [jnp.full((LANES,), -jnp.inf, jnp.float32)] * NCHAINS,
            [jnp.zeros((LANES,), jnp.int32)] * NCHAINS,
            jnp.zeros((LANES,), jnp.int32),
        )
        vmax, vchunk, _ = lax.fori_loop(0, QCHUNKS // UNROLL, chunk, init)
        bm = vmax[0]
        bi = vchunk[0] * LANES + iota
        for q in range(1, NCHAINS):
            vidx = vchunk[q] * LANES + (q * QCOLS) + iota
            p = vmax[q] > bm
            bm = jnp.where(p, vmax[q], bm)
            bi = jnp.where(p, vidx, bi)
        m = jnp.max(bm)
        cand = jnp.where(bm == m, bi, jnp.int32(1 << 30))
        imins.append(jnp.min(cand))

    ans_copy.wait()
    partial = jnp.zeros((LANES,), jnp.float32)
    for r in range(ROWS_PER_W):
        imin = imins[r]
        col0 = pl.multiple_of(jnp.bitwise_and(imin, jnp.int32(-LANES)), LANES)
        av = a_v[r, pl.ds(col0, LANES)]
        lane = jnp.bitwise_and(imin, jnp.int32(LANES - 1))
        partial = partial + jnp.where(iota == lane, av, jnp.float32(0.0))

    st_v[...] = partial
    pltpu.sync_copy(st_v, out_hbm.at[wid])


@jax.jit
def _sc_partials(score, ans_idx):
    mesh = plsc.VectorSubcoreMesh(core_axis_name="c", subcore_axis_name="s")
    return pl.kernel(
        _sc_body,
        out_type=jax.ShapeDtypeStruct((NUM_WORKERS, LANES), jnp.float32),
        mesh=mesh,
        scratch_types=[
            pltpu.VMEM((ROWS_PER_W, ROW), jnp.float32),
            pltpu.VMEM((ROWS_PER_W, ROW), jnp.float32),
            pltpu.VMEM((LANES,), jnp.float32),
            pltpu.SemaphoreType.DMA,
        ],
        compiler_params=pltpu.CompilerParams(needs_layout_passes=False),
    )(score, ans_idx)


def _reduce_body(p_ref, o_ref):
    o_ref[0, 0] = jnp.sum(p_ref[...]) * (100.0 / BATCH)


@jax.jit
def _tc_reduce(partials):
    return pl.pallas_call(
        _reduce_body,
        out_shape=jax.ShapeDtypeStruct((1, 1), jnp.float32),
        in_specs=[pl.BlockSpec(memory_space=pltpu.VMEM)],
        out_specs=pl.BlockSpec(memory_space=pltpu.SMEM),
    )(partials)


def kernel(score, ans_idx):
    partials = _sc_partials(score, ans_idx)
    acc = _tc_reduce(partials)
    return acc[0, 0]
